# Initial kernel scaffold; baseline (speedup 1.0000x reference)
import functools

import jax
import jax.numpy as jnp
from jax import lax
from jax.experimental import pallas as pl
from jax.experimental.pallas import tpu as pltpu
from jax.experimental.pallas import tpu_sc as plsc

VOCAB = 100000
D = 128
B = 4096
S = 200

NC = 2
NS = 16
NW = NC * NS
BPW = B // NW
LANES = 16
ROWV = D // LANES


def _body(idx_hbm, len_hbm, table_hbm, out_hbm, idx_v, len_v, acc, sem):
    wid = lax.axis_index("s") * NC + lax.axis_index("c")
    base = wid * BPW

    pltpu.sync_copy(idx_hbm.at[wid], idx_v)
    pltpu.sync_copy(len_hbm.at[pl.ds(base, BPW)], len_v)

    def zero(i, _):
        for j in range(ROWV):
            acc[i, pl.ds(j * LANES, LANES)] = jnp.zeros((LANES,), jnp.float32)
        return 0
    lax.fori_loop(0, BPW, zero, 0)

    def fire(s, _):
        pltpu.async_copy(table_hbm.at[idx_v.at[s]], acc, sem, add=True)
        return 0
    lax.fori_loop(0, S, fire, 0)

    def drain(s, _):
        pltpu.make_async_copy(table_hbm.at[idx_v.at[0]], acc, sem).wait()
        return 0
    lax.fori_loop(0, S, drain, 0)

    def scale(i, _):
        lv = plsc.load_gather(len_v, [jnp.full((LANES,), i, jnp.int32)])
        lf = lv.astype(jnp.float32)
        for j in range(ROWV):
            acc[i, pl.ds(j * LANES, LANES)] = acc[i, pl.ds(j * LANES, LANES)] / lf
        return 0
    lax.fori_loop(0, BPW, scale, 0)

    pltpu.sync_copy(acc, out_hbm.at[pl.ds(base, BPW)])


@jax.jit
def _run(idx_r, lengths, table):
    mesh = plsc.VectorSubcoreMesh(
        core_axis_name="c", subcore_axis_name="s",
        num_cores=NC, num_subcores=NS)
    f = functools.partial(
        pl.kernel,
        out_type=jax.ShapeDtypeStruct((B, D), jnp.float32),
        mesh=mesh,
        scratch_types=[
            pltpu.VMEM((S, BPW), jnp.int32),
            pltpu.VMEM((BPW,), jnp.int32),
            pltpu.VMEM((BPW, D), jnp.float32),
            pltpu.SemaphoreType.DMA,
        ],
    )(_body)
    return f(idx_r, lengths, table)


def kernel(indices, lengths, word_embeddings):
    idx_r = indices.reshape(NW, BPW, S).transpose(0, 2, 1)
    return _run(idx_r, lengths, word_embeddings)


# speedup vs baseline: 16.6614x; 16.6614x over previous
import functools

import jax
import jax.numpy as jnp
from jax import lax
from jax.experimental import pallas as pl
from jax.experimental.pallas import tpu as pltpu
from jax.experimental.pallas import tpu_sc as plsc

VOCAB = 100000
D = 128
B = 4096
S = 200

NC = 2
NS = 16
NW = NC * NS
BPW = B // NW
LANES = 16
ROWV = D // LANES


def _body(idx_hbm, len_hbm, table_hbm, out_hbm, idx_v, len_v, acc, sem):
    wid = lax.axis_index("s") * NC + lax.axis_index("c")
    base = wid * BPW

    pltpu.sync_copy(idx_hbm.at[wid], idx_v)
    pltpu.sync_copy(len_hbm.at[pl.ds(base, BPW)], len_v)


    def zero(i, _):
        for j in range(ROWV):
            acc[i, pl.ds(j * LANES, LANES)] = jnp.zeros((LANES,), jnp.float32)
        return 0
    lax.fori_loop(0, BPW, zero, 0)

    def fire(s, _):
        pltpu.async_copy(table_hbm.at[idx_v.at[s]], acc, sem, add=True)
        return 0
    lax.fori_loop(0, S, fire, 0)

    def drain(s, _):
        pltpu.make_async_copy(table_hbm.at[idx_v.at[0]], acc, sem).wait()
        return 0
    lax.fori_loop(0, S, drain, 0)

    def scale(i, _):
        lf = len_v[i]
        for j in range(ROWV):
            acc[i, pl.ds(j * LANES, LANES)] = acc[i, pl.ds(j * LANES, LANES)] / lf
        return 0
    lax.fori_loop(0, BPW, scale, 0)

    pltpu.sync_copy(acc, out_hbm.at[pl.ds(base, BPW)])


@jax.jit
def _run(idx_r, lengths, table):
    mesh = plsc.VectorSubcoreMesh(
        core_axis_name="c", subcore_axis_name="s",
        num_cores=NC, num_subcores=NS)
    f = functools.partial(
        pl.kernel,
        out_type=jax.ShapeDtypeStruct((B, D), jnp.float32),
        mesh=mesh,
        scratch_types=[
            pltpu.VMEM((S, BPW), jnp.int32),
            pltpu.VMEM((BPW, LANES), jnp.float32),
            pltpu.VMEM((BPW, D), jnp.float32),
            pltpu.SemaphoreType.DMA,
        ],
    )(_body)
    return f(idx_r, lengths, table)


def kernel(indices, lengths, word_embeddings):
    idx_r = indices.reshape(NW, BPW, S).transpose(0, 2, 1)
    len_b = jnp.broadcast_to(
        lengths.astype(jnp.float32)[:, None], (B, LANES))
    return _run(idx_r, len_b, word_embeddings)
